# baseline (speedup 1.0000x reference)
import functools

import jax
import jax.numpy as jnp
from jax import lax
from jax.experimental import pallas as pl
from jax.experimental.pallas import tpu as pltpu
from jax.experimental.pallas import tpu_sc as plsc

SEQ = 2048
HIDDEN = 64
VOCAB = 512
B = SEQ * SEQ
NW = 32
LOOK_PER_W = B // NW
ROW = 128
CH = 512
KS = 2
CHS = KS * ROW
CHV = CH - CHS
NG = LOOK_PER_W // CH
NROWS = B // ROW
NBUF = 2
L = 16


def _make_gather():
    mesh = plsc.VectorSubcoreMesh(core_axis_name="c", subcore_axis_name="s")

    @functools.partial(
        pl.kernel,
        mesh=mesh,
        out_type=jax.ShapeDtypeStruct((B, HIDDEN), jnp.float32),
        scratch_types=[
            pltpu.VMEM((VOCAB * HIDDEN + 8,), jnp.float32),
            pltpu.VMEM_SHARED((VOCAB, HIDDEN), jnp.float32),
            pltpu.VMEM((KS, ROW), jnp.int32),
            pltpu.VMEM((KS, ROW), jnp.int32),
            pltpu.VMEM((CHV // ROW, ROW), jnp.int32),
            pltpu.VMEM((CHV // ROW, ROW), jnp.int32),
            pltpu.VMEM((CHS, HIDDEN), jnp.float32),
            pltpu.VMEM((CHS, HIDDEN), jnp.float32),
            pltpu.VMEM((CHV, HIDDEN), jnp.float32),
            pltpu.VMEM((CHV, HIDDEN), jnp.float32),
            pltpu.SemaphoreType.DMA,
            pltpu.SemaphoreType.DMA,
            pltpu.SemaphoreType.DMA,
            pltpu.SemaphoreType.DMA,
            pltpu.SemaphoreType.DMA,
            pltpu.SemaphoreType.DMA,
        ],
        compiler_params=pltpu.CompilerParams(
            use_tc_tiling_on_sc=False, needs_layout_passes=False),
    )
    def gather_kernel(table_hbm, tablef_hbm, idx2_hbm, out_hbm,
                      table_v, table_sp,
                      idxs0, idxs1, idxv0, idxv1,
                      sstage0, sstage1, vstage0, vstage1,
                      si0, si1, sg0, sg1, so0, so1):
        idxs_bufs = (idxs0, idxs1)
        idxv_bufs = (idxv0, idxv1)
        sstage = (sstage0, sstage1)
        vstage = (vstage0, vstage1)
        sem_i = (si0, si1)
        sem_g = (sg0, sg1)
        sem_o = (so0, so1)

        c = lax.axis_index("c")
        s = lax.axis_index("s")
        wid = s * 2 + c
        base_look = wid * LOOK_PER_W
        base_row = base_look // ROW

        @pl.when(s == 0)
        def _():
            pltpu.sync_copy(table_hbm, table_sp)
        pltpu.sync_copy(tablef_hbm, table_v)
        plsc.subcore_barrier()

        def idx_start(g, p):
            row0 = base_row + g * (CH // ROW)
            pltpu.async_copy(
                idx2_hbm.at[pl.ds(row0, KS)], idxs_bufs[p], sem_i[p])
            pltpu.async_copy(
                idx2_hbm.at[pl.ds(row0 + KS, CHV // ROW)],
                idxv_bufs[p], sem_i[p])

        def idx_wait(p):
            pltpu.make_async_copy(
                idx2_hbm.at[pl.ds(0, KS)], idxs_bufs[p], sem_i[p]).wait()
            pltpu.make_async_copy(
                idx2_hbm.at[pl.ds(0, CHV // ROW)],
                idxv_bufs[p], sem_i[p]).wait()

        def out_descs(g, p):
            base = base_look + g * CH
            return (
                pltpu.make_async_copy(
                    sstage[p], out_hbm.at[pl.ds(base, CHS)], sem_o[p]),
                pltpu.make_async_copy(
                    vstage[p], out_hbm.at[pl.ds(base + CHS, CHV)], sem_o[p]),
            )

        idx_start(0, 0)

        lane_iota = lax.iota(jnp.int32, L)
        dst_iota = lane_iota * HIDDEN

        def group(g, p):
            @pl.when(g >= NBUF)
            def _():
                for d in out_descs(g - NBUF, p):
                    d.wait()

            idx_wait(p)

            @pl.when(g + 1 < NG)
            def _():
                idx_start(g + 1, 1 - p)

            gathers = [
                pltpu.async_copy(
                    table_sp.at[idxs_bufs[p].at[j]],
                    sstage[p].at[pl.ds(j * ROW, ROW)],
                    sem_g[p])
                for j in range(KS)
            ]

            for r in range(CHV // ROW):
                @plsc.parallel_loop(0, ROW // L, unroll=2)
                def b_body(b, r=r):
                    idx16 = idxv_bufs[p].at[r][pl.ds(b * L, L)]
                    src_base = idx16 * HIDDEN
                    dst_rows = (r * ROW + b * L) + lane_iota
                    for cb in range(0, HIDDEN, 8):
                        sks = [lane_iota ^ (cb + j) for j in range(8)]
                        vals = [
                            plsc.load_gather(table_v, [src_base + sks[j]])
                            for j in range(8)
                        ]
                        for j in range(8):
                            plsc.store_scatter(
                                vstage[p], [dst_rows, sks[j]], vals[j])

            for cp in gathers:
                cp.wait()

            base = base_look + g * CH
            pltpu.async_copy(
                sstage[p], out_hbm.at[pl.ds(base, CHS)], sem_o[p])
            pltpu.async_copy(
                vstage[p], out_hbm.at[pl.ds(base + CHS, CHV)], sem_o[p])

        def outer(gg, carry):
            for p in range(NBUF):
                group(gg * NBUF + p, p)
            return carry

        lax.fori_loop(0, NG // NBUF, outer, 0)

        for p in range(NBUF):
            for d in out_descs(NG - NBUF + p, p):
                d.wait()

    return gather_kernel


_gather = _make_gather()


def kernel(dist_mat, table):
    idx2 = dist_mat.astype(jnp.int32).reshape(NROWS, ROW)
    tablef = jnp.concatenate(
        [table.reshape(VOCAB * HIDDEN), jnp.zeros((8,), jnp.float32)])
    out = _gather(table, tablef, idx2)
    return out.reshape(SEQ, SEQ, HIDDEN)


# speedup vs baseline: 1.3515x; 1.3515x over previous
import functools

import jax
import jax.numpy as jnp
from jax import lax
from jax.experimental import pallas as pl
from jax.experimental.pallas import tpu as pltpu
from jax.experimental.pallas import tpu_sc as plsc

SEQ = 2048
HIDDEN = 64
VOCAB = 512
B = SEQ * SEQ
ROW = 128
NROWS = B // ROW
NW = 32
ROWS_PER_W = NROWS // NW
K = 4
NG = ROWS_PER_W // K
NBUF = 2


def _make_gather():
    mesh = plsc.VectorSubcoreMesh(core_axis_name="c", subcore_axis_name="s")

    @functools.partial(
        pl.kernel,
        mesh=mesh,
        out_type=jax.ShapeDtypeStruct((B, HIDDEN), jnp.float32),
        scratch_types=[
            pltpu.VMEM((K, ROW), jnp.int32),
            pltpu.VMEM((K, ROW), jnp.int32),
            pltpu.VMEM((K * ROW, HIDDEN), jnp.float32),
            pltpu.VMEM((K * ROW, HIDDEN), jnp.float32),
            pltpu.VMEM_SHARED((VOCAB, HIDDEN), jnp.float32),
            pltpu.SemaphoreType.DMA,
            pltpu.SemaphoreType.DMA,
            pltpu.SemaphoreType.DMA,
            pltpu.SemaphoreType.DMA,
            pltpu.SemaphoreType.DMA,
            pltpu.SemaphoreType.DMA,
        ],
        compiler_params=pltpu.CompilerParams(use_tc_tiling_on_sc=False),
    )
    def gather_kernel(table_hbm, idx_hbm, out_hbm,
                      idx_v0, idx_v1, rows_v0, rows_v1, table_sp,
                      si0, si1, sg0, sg1, so0, so1):
        idx_bufs = (idx_v0, idx_v1)
        rows_bufs = (rows_v0, rows_v1)
        sem_i = (si0, si1)
        sem_g = (sg0, sg1)
        sem_o = (so0, so1)

        c = lax.axis_index("c")
        s = lax.axis_index("s")
        wid = s * 2 + c
        base_row = wid * ROWS_PER_W

        @pl.when(s == 0)
        def _():
            pltpu.sync_copy(table_hbm, table_sp)

        plsc.subcore_barrier()

        def idx_start(g, p):
            pltpu.async_copy(
                idx_hbm.at[pl.ds(base_row + g * K, K)], idx_bufs[p], sem_i[p])

        def idx_wait(p):
            pltpu.make_async_copy(
                idx_hbm.at[pl.ds(0, K)], idx_bufs[p], sem_i[p]).wait()

        def gather_start(p):
            for j in range(K):
                pltpu.async_copy(
                    table_sp.at[idx_bufs[p].at[j]],
                    rows_bufs[p].at[pl.ds(j * ROW, ROW)],
                    sem_g[p])

        def gather_wait(p):
            for j in range(K):
                pltpu.make_async_copy(
                    table_sp.at[idx_bufs[p].at[j]],
                    rows_bufs[p].at[pl.ds(j * ROW, ROW)],
                    sem_g[p]).wait()

        def out_start(g, p):
            pltpu.async_copy(
                rows_bufs[p],
                out_hbm.at[pl.ds((base_row + g * K) * ROW, K * ROW)],
                sem_o[p])

        def out_wait(g, p):
            pltpu.make_async_copy(
                rows_bufs[p],
                out_hbm.at[pl.ds((base_row + g * K) * ROW, K * ROW)],
                sem_o[p]).wait()

        idx_start(0, 0)
        idx_start(1, 1)
        idx_wait(0)
        gather_start(0)

        def group(g, p):
            @pl.when(g + 1 < NG)
            def _():
                idx_wait(1 - p)

                @pl.when(g >= 1)
                def _():
                    out_wait(g - 1, 1 - p)

                gather_start(1 - p)

            gather_wait(p)
            out_start(g, p)

            @pl.when(g + 2 < NG)
            def _():
                idx_start(g + 2, p)

        def outer(gg, carry):
            for p in range(NBUF):
                group(gg * NBUF + p, p)
            return carry

        lax.fori_loop(0, NG // NBUF, outer, 0)

        out_wait(NG - 2, 0)
        out_wait(NG - 1, 1)

    return gather_kernel


_gather = _make_gather()


def kernel(dist_mat, table):
    idx = dist_mat.astype(jnp.int32).reshape(NROWS, ROW)
    out = _gather(table, idx)
    return out.reshape(SEQ, SEQ, HIDDEN)


# speedup vs baseline: 1.3517x; 1.0001x over previous
import functools

import jax
import jax.numpy as jnp
from jax import lax
from jax.experimental import pallas as pl
from jax.experimental.pallas import tpu as pltpu
from jax.experimental.pallas import tpu_sc as plsc

SEQ = 2048
HIDDEN = 64
VOCAB = 512
B = SEQ * SEQ
ROW = 128
NROWS = B // ROW
NW = 32
ROWS_PER_W = NROWS // NW
K = 4
NG = ROWS_PER_W // K
NBUF = 2


def _make_gather():
    mesh = plsc.VectorSubcoreMesh(core_axis_name="c", subcore_axis_name="s")

    @functools.partial(
        pl.kernel,
        mesh=mesh,
        out_type=jax.ShapeDtypeStruct((B, HIDDEN), jnp.float32),
        scratch_types=[
            pltpu.VMEM((K, ROW), jnp.int32),
            pltpu.VMEM((K, ROW), jnp.int32),
            pltpu.VMEM((K * ROW, HIDDEN), jnp.float32),
            pltpu.VMEM((K * ROW, HIDDEN), jnp.float32),
            pltpu.VMEM_SHARED((VOCAB, HIDDEN), jnp.float32),
            pltpu.SemaphoreType.DMA,
            pltpu.SemaphoreType.DMA,
            pltpu.SemaphoreType.DMA,
            pltpu.SemaphoreType.DMA,
            pltpu.SemaphoreType.DMA,
            pltpu.SemaphoreType.DMA,
        ],
        compiler_params=pltpu.CompilerParams(
            use_tc_tiling_on_sc=False, disable_bounds_checks=True),
    )
    def gather_kernel(table_hbm, idx_hbm, out_hbm,
                      idx_v0, idx_v1, rows_v0, rows_v1, table_sp,
                      si0, si1, sg0, sg1, so0, so1):
        idx_bufs = (idx_v0, idx_v1)
        rows_bufs = (rows_v0, rows_v1)
        sem_i = (si0, si1)
        sem_g = (sg0, sg1)
        sem_o = (so0, so1)

        c = lax.axis_index("c")
        s = lax.axis_index("s")
        wid = s * 2 + c
        base_row = wid * ROWS_PER_W

        @pl.when(s == 0)
        def _():
            pltpu.sync_copy(table_hbm, table_sp)

        plsc.subcore_barrier()

        def idx_start(g, p):
            pltpu.async_copy(
                idx_hbm.at[pl.ds(base_row + g * K, K)], idx_bufs[p], sem_i[p])

        def idx_wait(p):
            pltpu.make_async_copy(
                idx_hbm.at[pl.ds(0, K)], idx_bufs[p], sem_i[p]).wait()

        def gather_start(p):
            for j in range(K):
                pltpu.async_copy(
                    table_sp.at[idx_bufs[p].at[j]],
                    rows_bufs[p].at[pl.ds(j * ROW, ROW)],
                    sem_g[p])

        def gather_wait(p):
            for j in range(K):
                pltpu.make_async_copy(
                    table_sp.at[idx_bufs[p].at[j]],
                    rows_bufs[p].at[pl.ds(j * ROW, ROW)],
                    sem_g[p]).wait()

        def out_start(g, p):
            pltpu.async_copy(
                rows_bufs[p],
                out_hbm.at[pl.ds((base_row + g * K) * ROW, K * ROW)],
                sem_o[p])

        def out_wait(g, p):
            pltpu.make_async_copy(
                rows_bufs[p],
                out_hbm.at[pl.ds((base_row + g * K) * ROW, K * ROW)],
                sem_o[p]).wait()

        idx_start(0, 0)
        idx_start(1, 1)
        idx_wait(0)
        gather_start(0)

        def group(g, p):
            @pl.when(g + 1 < NG)
            def _():
                idx_wait(1 - p)

                @pl.when(g >= 1)
                def _():
                    out_wait(g - 1, 1 - p)

                gather_start(1 - p)

            gather_wait(p)
            out_start(g, p)

            @pl.when(g + 2 < NG)
            def _():
                idx_start(g + 2, p)

        def outer(gg, carry):
            for p in range(NBUF):
                group(gg * NBUF + p, p)
            return carry

        lax.fori_loop(0, NG // NBUF, outer, 0)

        out_wait(NG - 2, 0)
        out_wait(NG - 1, 1)

    return gather_kernel


_gather = _make_gather()


def kernel(dist_mat, table):
    idx = dist_mat.astype(jnp.int32).reshape(NROWS, ROW)
    out = _gather(table, idx)
    return out.reshape(SEQ, SEQ, HIDDEN)
